# baseline (speedup 1.0000x reference)
import functools
import jax
import jax.numpy as jnp
from jax import lax
from jax.experimental import pallas as pl
from jax.experimental.pallas import tpu as pltpu
from jax.experimental.pallas import tpu_sc as plsc

NC, NS, L = 2, 16, 16
NW = NC * NS
EMBED_DIM = 64
SCALE = 8.0

N_ROWS = 4096
ROW_W = 200
ROWS_PER_W = N_ROWS // NW
SPLIT = (128, 72)
NBUF = 2


def _make_kernel():
    mesh = plsc.VectorSubcoreMesh(
        core_axis_name="c", subcore_axis_name="s", num_cores=NC, num_subcores=NS
    )

    @functools.partial(
        pl.kernel,
        out_type=jax.ShapeDtypeStruct((N_ROWS, ROW_W, EMBED_DIM), jnp.float32),
        mesh=mesh,
        scratch_types=[
            pltpu.VMEM((ROWS_PER_W, ROW_W), jnp.int32),
            pltpu.VMEM((NBUF, ROW_W, EMBED_DIM), jnp.float32),
            pltpu.SemaphoreType.DMA,
            pltpu.SemaphoreType.DMA,
            pltpu.SemaphoreType.DMA,
            pltpu.SemaphoreType.DMA,
        ],
        compiler_params=pltpu.CompilerParams(use_tc_tiling_on_sc=False),
    )
    def emb_kernel(tokens_hbm, table_hbm, out_hbm, idx_v, rows_v, g0, g1, s0, s1):
        wid = lax.axis_index("s") * NC + lax.axis_index("c")
        gsem = (g0, g1)
        ssem = (s0, s1)
        row0 = wid * ROWS_PER_W
        pltpu.sync_copy(tokens_hbm.at[pl.ds(row0, ROWS_PER_W)], idx_v)

        def fire_gathers(b, r):
            off = 0
            for w in SPLIT:
                pltpu.async_copy(
                    table_hbm.at[idx_v.at[r, pl.ds(off, w)]],
                    rows_v.at[b, pl.ds(off, w)],
                    gsem[b],
                )
                off += w

        def drain_gathers(b):
            off = 0
            for w in SPLIT:
                pltpu.make_async_copy(
                    table_hbm.at[idx_v.at[0, pl.ds(off, w)]],
                    rows_v.at[b, pl.ds(off, w)],
                    gsem[b],
                ).wait()
                off += w

        def scale_buf(b):
            @pl.loop(0, ROW_W, unroll=4)
            def _(i):
                for t in range(EMBED_DIM // L):
                    sl = pl.ds(t * L, L)
                    rows_v[b, i, sl] = rows_v[b, i, sl] * SCALE

        def wait_store(b):
            pltpu.make_async_copy(rows_v.at[b], out_hbm.at[0], ssem[b]).wait()

        fire_gathers(0, 0)

        @pl.loop(0, ROWS_PER_W // NBUF)
        def _(o):
            for b in range(NBUF):
                r = o * NBUF + b
                nb = (b + 1) % NBUF

                @pl.when(r + 1 < ROWS_PER_W)
                def _():
                    @pl.when(r >= 1)
                    def _():
                        wait_store(nb)

                    fire_gathers(nb, r + 1)

                drain_gathers(b)
                scale_buf(b)
                pltpu.async_copy(rows_v.at[b], out_hbm.at[row0 + r], ssem[b])

        wait_store(0)
        wait_store(1)

    return emb_kernel


_emb_kernel = _make_kernel()


@jax.jit
def kernel(tokens, table):
    return _emb_kernel(tokens.astype(jnp.int32), table)
